# Initial kernel scaffold; baseline (speedup 1.0000x reference)
import jax
import jax.numpy as jnp
from jax.experimental import pallas as pl


def kernel(x, adj_p, w1_p, b1_p, w2_p, b2_p):
    raise NotImplementedError("write your pallas kernel here")


# speedup vs baseline: 2.9143x; 2.9143x over previous
import functools

import jax
import jax.numpy as jnp
from jax.experimental import pallas as pl
from jax.experimental.pallas import tpu as pltpu


def _support1_kernel(x_ref, w1_ref, o_ref):
    x = x_ref[...].astype(jnp.bfloat16)
    o_ref[...] = jnp.dot(
        x, w1_ref[...], preferred_element_type=jnp.float32
    ).astype(o_ref.dtype)


def _layer1_kernel(s1_ref, adj_ref, b1_ref, w2_ref, o_ref):
    acc = jnp.dot(adj_ref[...], s1_ref[...], preferred_element_type=jnp.float32)
    h = jnp.maximum(acc + b1_ref[...], 0.0).astype(jnp.bfloat16)
    o_ref[...] = jnp.dot(
        h, w2_ref[...], preferred_element_type=jnp.float32
    ).astype(o_ref.dtype)


def _layer2_kernel(s2_ref, adj_ref, b2_ref, o_ref):
    acc = jnp.dot(adj_ref[...], s2_ref[...], preferred_element_type=jnp.float32)
    o_ref[...] = acc + b2_ref[...]


@jax.jit
def _forward(x, adj_p, w1_p, b1_p, w2_p, b2_p):
    Np = adj_p.shape[0]
    D = x.shape[1]
    Z = w1_p.shape[1]
    C = w2_p.shape[1]
    cd = jnp.bfloat16

    TM1 = 1024
    TM = 512

    s1 = pl.pallas_call(
        _support1_kernel,
        out_shape=jax.ShapeDtypeStruct((Np, Z), cd),
        grid_spec=pltpu.PrefetchScalarGridSpec(
            num_scalar_prefetch=0,
            grid=(Np // TM1,),
            in_specs=[
                pl.BlockSpec((TM1, D), lambda i: (i, 0)),
                pl.BlockSpec((D, Z), lambda i: (0, 0)),
            ],
            out_specs=pl.BlockSpec((TM1, Z), lambda i: (i, 0)),
        ),
        compiler_params=pltpu.CompilerParams(dimension_semantics=("parallel",)),
    )(x, w1_p)

    s2 = pl.pallas_call(
        _layer1_kernel,
        out_shape=jax.ShapeDtypeStruct((Np, C), cd),
        grid_spec=pltpu.PrefetchScalarGridSpec(
            num_scalar_prefetch=0,
            grid=(Np // TM,),
            in_specs=[
                pl.BlockSpec((Np, Z), lambda i: (0, 0)),
                pl.BlockSpec((TM, Np), lambda i: (i, 0)),
                pl.BlockSpec((1, Z), lambda i: (0, 0)),
                pl.BlockSpec((Z, C), lambda i: (0, 0)),
            ],
            out_specs=pl.BlockSpec((TM, C), lambda i: (i, 0)),
        ),
        compiler_params=pltpu.CompilerParams(dimension_semantics=("parallel",)),
    )(s1, adj_p, b1_p, w2_p)

    out = pl.pallas_call(
        _layer2_kernel,
        out_shape=jax.ShapeDtypeStruct((Np, C), jnp.float32),
        grid_spec=pltpu.PrefetchScalarGridSpec(
            num_scalar_prefetch=0,
            grid=(Np // TM,),
            in_specs=[
                pl.BlockSpec((Np, C), lambda i: (0, 0)),
                pl.BlockSpec((TM, Np), lambda i: (i, 0)),
                pl.BlockSpec((1, C), lambda i: (0, 0)),
            ],
            out_specs=pl.BlockSpec((TM, C), lambda i: (i, 0)),
        ),
        compiler_params=pltpu.CompilerParams(dimension_semantics=("parallel",)),
    )(s2, adj_p, b2_p)

    return out


def kernel(x, adj_p, w1_p, b1_p, w2_p, b2_p):
    N = x.shape[0]
    C = w2_p.shape[1]
    out = _forward(x, adj_p, w1_p, b1_p, w2_p, b2_p)
    return out[:N, :C]


# speedup vs baseline: 2.9576x; 1.0149x over previous
import functools

import jax
import jax.numpy as jnp
from jax.experimental import pallas as pl
from jax.experimental.pallas import tpu as pltpu


def _support1_kernel(x_ref, w1_ref, o_ref):
    x = x_ref[...].astype(jnp.bfloat16)
    o_ref[...] = jnp.dot(
        x, w1_ref[...], preferred_element_type=jnp.float32
    ).astype(o_ref.dtype)


def _layer1_kernel(s1_ref, adj_ref, b1_ref, w2_ref, o_ref):
    acc = jnp.dot(adj_ref[...], s1_ref[...], preferred_element_type=jnp.float32)
    h = jnp.maximum(acc + b1_ref[...], 0.0).astype(jnp.bfloat16)
    o_ref[...] = jnp.dot(
        h, w2_ref[...], preferred_element_type=jnp.float32
    ).astype(o_ref.dtype)


def _layer2_kernel(s2_ref, adj_ref, b2_ref, o_ref):
    acc = jnp.dot(adj_ref[...], s2_ref[...], preferred_element_type=jnp.float32)
    o_ref[...] = acc + b2_ref[...]


@jax.jit
def _forward(x, adj_p, w1_p, b1_p, w2_p, b2_p):
    Np = adj_p.shape[0]
    D = x.shape[1]
    Z = w1_p.shape[1]
    C = w2_p.shape[1]
    cd = jnp.bfloat16

    TM1 = 1024
    TM = 512
    TM2 = 1024

    s1 = pl.pallas_call(
        _support1_kernel,
        out_shape=jax.ShapeDtypeStruct((Np, Z), cd),
        grid_spec=pltpu.PrefetchScalarGridSpec(
            num_scalar_prefetch=0,
            grid=(Np // TM1,),
            in_specs=[
                pl.BlockSpec((TM1, D), lambda i: (i, 0)),
                pl.BlockSpec((D, Z), lambda i: (0, 0)),
            ],
            out_specs=pl.BlockSpec((TM1, Z), lambda i: (i, 0)),
        ),
        compiler_params=pltpu.CompilerParams(dimension_semantics=("parallel",)),
    )(x, w1_p)

    s2 = pl.pallas_call(
        _layer1_kernel,
        out_shape=jax.ShapeDtypeStruct((Np, C), cd),
        grid_spec=pltpu.PrefetchScalarGridSpec(
            num_scalar_prefetch=0,
            grid=(Np // TM,),
            in_specs=[
                pl.BlockSpec((Np, Z), lambda i: (0, 0)),
                pl.BlockSpec((TM, Np), lambda i: (i, 0)),
                pl.BlockSpec((1, Z), lambda i: (0, 0)),
                pl.BlockSpec((Z, C), lambda i: (0, 0)),
            ],
            out_specs=pl.BlockSpec((TM, C), lambda i: (i, 0)),
        ),
        compiler_params=pltpu.CompilerParams(dimension_semantics=("parallel",)),
    )(s1, adj_p, b1_p, w2_p)

    out = pl.pallas_call(
        _layer2_kernel,
        out_shape=jax.ShapeDtypeStruct((Np, C), jnp.float32),
        grid_spec=pltpu.PrefetchScalarGridSpec(
            num_scalar_prefetch=0,
            grid=(Np // TM2,),
            in_specs=[
                pl.BlockSpec((Np, C), lambda i: (0, 0)),
                pl.BlockSpec((TM2, Np), lambda i: (i, 0)),
                pl.BlockSpec((1, C), lambda i: (0, 0)),
            ],
            out_specs=pl.BlockSpec((TM2, C), lambda i: (i, 0)),
        ),
        compiler_params=pltpu.CompilerParams(dimension_semantics=("parallel",)),
    )(s2, adj_p, b2_p)

    return out


def kernel(x, adj_p, w1_p, b1_p, w2_p, b2_p):
    N = x.shape[0]
    C = w2_p.shape[1]
    out = _forward(x, adj_p, w1_p, b1_p, w2_p, b2_p)
    return out[:N, :C]


# speedup vs baseline: 2.9716x; 1.0048x over previous
import functools

import jax
import jax.numpy as jnp
from jax.experimental import pallas as pl
from jax.experimental.pallas import tpu as pltpu


def _support1_kernel(x_ref, w1_ref, o_ref):
    x = x_ref[...].astype(jnp.bfloat16)
    o_ref[...] = jnp.dot(
        x, w1_ref[...], preferred_element_type=jnp.float32
    ).astype(o_ref.dtype)


def _layer1_kernel(s1_ref, adj_ref, b1_ref, w2_ref, o_ref):
    acc = jnp.dot(adj_ref[...], s1_ref[...], preferred_element_type=jnp.float32)
    h = jnp.maximum(acc + b1_ref[...], 0.0).astype(jnp.bfloat16)
    o_ref[...] = jnp.dot(
        h, w2_ref[...], preferred_element_type=jnp.float32
    ).astype(o_ref.dtype)


def _layer2_kernel(s2_ref, adj_ref, b2_ref, o_ref):
    acc = jnp.dot(adj_ref[...], s2_ref[...], preferred_element_type=jnp.float32)
    o_ref[...] = acc + b2_ref[...]


@jax.jit
def _forward(x, adj_p, w1_p, b1_p, w2_p, b2_p):
    Np = adj_p.shape[0]
    D = x.shape[1]
    Z = w1_p.shape[1]
    C = w2_p.shape[1]
    cd = jnp.bfloat16

    TM1 = 1024
    TM = 1024
    TM2 = 1024

    s1 = pl.pallas_call(
        _support1_kernel,
        out_shape=jax.ShapeDtypeStruct((Np, Z), cd),
        grid_spec=pltpu.PrefetchScalarGridSpec(
            num_scalar_prefetch=0,
            grid=(Np // TM1,),
            in_specs=[
                pl.BlockSpec((TM1, D), lambda i: (i, 0)),
                pl.BlockSpec((D, Z), lambda i: (0, 0)),
            ],
            out_specs=pl.BlockSpec((TM1, Z), lambda i: (i, 0)),
        ),
        compiler_params=pltpu.CompilerParams(dimension_semantics=("parallel",)),
    )(x, w1_p)

    s2 = pl.pallas_call(
        _layer1_kernel,
        out_shape=jax.ShapeDtypeStruct((Np, C), cd),
        grid_spec=pltpu.PrefetchScalarGridSpec(
            num_scalar_prefetch=0,
            grid=(Np // TM,),
            in_specs=[
                pl.BlockSpec((Np, Z), lambda i: (0, 0)),
                pl.BlockSpec((TM, Np), lambda i: (i, 0)),
                pl.BlockSpec((1, Z), lambda i: (0, 0)),
                pl.BlockSpec((Z, C), lambda i: (0, 0)),
            ],
            out_specs=pl.BlockSpec((TM, C), lambda i: (i, 0)),
        ),
        compiler_params=pltpu.CompilerParams(dimension_semantics=("parallel",)),
    )(s1, adj_p, b1_p, w2_p)

    out = pl.pallas_call(
        _layer2_kernel,
        out_shape=jax.ShapeDtypeStruct((Np, C), jnp.float32),
        grid_spec=pltpu.PrefetchScalarGridSpec(
            num_scalar_prefetch=0,
            grid=(Np // TM2,),
            in_specs=[
                pl.BlockSpec((Np, C), lambda i: (0, 0)),
                pl.BlockSpec((TM2, Np), lambda i: (i, 0)),
                pl.BlockSpec((1, C), lambda i: (0, 0)),
            ],
            out_specs=pl.BlockSpec((TM2, C), lambda i: (i, 0)),
        ),
        compiler_params=pltpu.CompilerParams(dimension_semantics=("parallel",)),
    )(s2, adj_p, b2_p)

    return out


def kernel(x, adj_p, w1_p, b1_p, w2_p, b2_p):
    N = x.shape[0]
    C = w2_p.shape[1]
    out = _forward(x, adj_p, w1_p, b1_p, w2_p, b2_p)
    return out[:N, :C]


# speedup vs baseline: 2.9919x; 1.0068x over previous
import functools

import jax
import jax.numpy as jnp
from jax.experimental import pallas as pl
from jax.experimental.pallas import tpu as pltpu


def _support1_kernel(x_ref, w1_ref, o_ref):
    x = x_ref[...].astype(jnp.bfloat16)
    o_ref[...] = jnp.dot(
        x, w1_ref[...], preferred_element_type=jnp.float32
    ).astype(o_ref.dtype)


def _layer1_kernel(s1_ref, adj_ref, b1_ref, w2_ref, o_ref):
    acc = jnp.dot(adj_ref[...], s1_ref[...], preferred_element_type=jnp.float32)
    h = jnp.maximum(acc + b1_ref[...], 0.0).astype(jnp.bfloat16)
    o_ref[...] = jnp.dot(
        h, w2_ref[...], preferred_element_type=jnp.float32
    ).astype(o_ref.dtype)


def _layer2_kernel(s2_ref, adj_ref, b2_ref, o_ref):
    acc = jnp.dot(adj_ref[...], s2_ref[...], preferred_element_type=jnp.float32)
    o_ref[...] = acc + b2_ref[...]


@jax.jit
def _forward(x, adj_p, w1_p, b1_p, w2_p, b2_p):
    Np = adj_p.shape[0]
    D = x.shape[1]
    Z = w1_p.shape[1]
    C = w2_p.shape[1]
    cd = jnp.bfloat16

    TM1 = 2048
    TM = 1024
    TM2 = 1024

    s1 = pl.pallas_call(
        _support1_kernel,
        out_shape=jax.ShapeDtypeStruct((Np, Z), cd),
        grid_spec=pltpu.PrefetchScalarGridSpec(
            num_scalar_prefetch=0,
            grid=(Np // TM1,),
            in_specs=[
                pl.BlockSpec((TM1, D), lambda i: (i, 0)),
                pl.BlockSpec((D, Z), lambda i: (0, 0)),
            ],
            out_specs=pl.BlockSpec((TM1, Z), lambda i: (i, 0)),
        ),
        compiler_params=pltpu.CompilerParams(dimension_semantics=("parallel",)),
    )(x, w1_p)
    s2 = pl.pallas_call(
        _layer1_kernel,
        out_shape=jax.ShapeDtypeStruct((Np, C), cd),
        grid_spec=pltpu.PrefetchScalarGridSpec(
            num_scalar_prefetch=0,
            grid=(Np // TM,),
            in_specs=[
                pl.BlockSpec((Np, Z), lambda i: (0, 0)),
                pl.BlockSpec((TM, Np), lambda i: (i, 0)),
                pl.BlockSpec((1, Z), lambda i: (0, 0)),
                pl.BlockSpec((Z, C), lambda i: (0, 0)),
            ],
            out_specs=pl.BlockSpec((TM, C), lambda i: (i, 0)),
        ),
        compiler_params=pltpu.CompilerParams(dimension_semantics=("parallel",)),
    )(s1, adj_p, b1_p, w2_p)

    out = pl.pallas_call(
        _layer2_kernel,
        out_shape=jax.ShapeDtypeStruct((Np, C), jnp.float32),
        grid_spec=pltpu.PrefetchScalarGridSpec(
            num_scalar_prefetch=0,
            grid=(Np // TM2,),
            in_specs=[
                pl.BlockSpec((Np, C), lambda i: (0, 0)),
                pl.BlockSpec((TM2, Np), lambda i: (i, 0)),
                pl.BlockSpec((1, C), lambda i: (0, 0)),
            ],
            out_specs=pl.BlockSpec((TM2, C), lambda i: (i, 0)),
        ),
        compiler_params=pltpu.CompilerParams(dimension_semantics=("parallel",)),
    )(s2, adj_p, b2_p)

    return out


def kernel(x, adj_p, w1_p, b1_p, w2_p, b2_p):
    N = x.shape[0]
    C = w2_p.shape[1]
    out = _forward(x, adj_p, w1_p, b1_p, w2_p, b2_p)
    return out[:N, :C]


# speedup vs baseline: 3.0090x; 1.0057x over previous
import functools

import jax
import jax.numpy as jnp
from jax.experimental import pallas as pl
from jax.experimental.pallas import tpu as pltpu


_MRB_M = 512


def _support1_kernel(x_ref, w1_ref, o_ref):
    for m in range(x_ref.shape[0] // _MRB_M):
        sl = slice(m * _MRB_M, (m + 1) * _MRB_M)
        x = x_ref[sl, :].astype(jnp.bfloat16)
        o_ref[sl, :] = jnp.dot(
            x, w1_ref[...], preferred_element_type=jnp.float32
        ).astype(o_ref.dtype)


def _layer1_kernel(s1_ref, adj_ref, b1_ref, w2_ref, o_ref):
    for m in range(adj_ref.shape[0] // _MRB_M):
        sl = slice(m * _MRB_M, (m + 1) * _MRB_M)
        acc = jnp.dot(
            adj_ref[sl, :], s1_ref[...], preferred_element_type=jnp.float32
        )
        h = jnp.maximum(acc + b1_ref[...], 0.0).astype(jnp.bfloat16)
        o_ref[sl, :] = jnp.dot(
            h, w2_ref[...], preferred_element_type=jnp.float32
        ).astype(o_ref.dtype)


def _layer2_kernel(s2_ref, adj_ref, b2_ref, o_ref):
    acc = jnp.dot(adj_ref[...], s2_ref[...], preferred_element_type=jnp.float32)
    o_ref[...] = acc + b2_ref[...]


@jax.jit
def _forward(x, adj_p, w1_p, b1_p, w2_p, b2_p):
    Np = adj_p.shape[0]
    D = x.shape[1]
    Z = w1_p.shape[1]
    C = w2_p.shape[1]
    cd = jnp.bfloat16

    TM1 = 2048
    TM = 1024
    TM2 = 1024

    s1 = pl.pallas_call(
        _support1_kernel,
        out_shape=jax.ShapeDtypeStruct((Np, Z), cd),
        grid_spec=pltpu.PrefetchScalarGridSpec(
            num_scalar_prefetch=0,
            grid=(Np // TM1,),
            in_specs=[
                pl.BlockSpec((TM1, D), lambda i: (i, 0)),
                pl.BlockSpec((D, Z), lambda i: (0, 0)),
            ],
            out_specs=pl.BlockSpec((TM1, Z), lambda i: (i, 0)),
        ),
        compiler_params=pltpu.CompilerParams(dimension_semantics=("parallel",)),
    )(x, w1_p)
    s2 = pl.pallas_call(
        _layer1_kernel,
        out_shape=jax.ShapeDtypeStruct((Np, C), cd),
        grid_spec=pltpu.PrefetchScalarGridSpec(
            num_scalar_prefetch=0,
            grid=(Np // TM,),
            in_specs=[
                pl.BlockSpec((Np, Z), lambda i: (0, 0)),
                pl.BlockSpec((TM, Np), lambda i: (i, 0)),
                pl.BlockSpec((1, Z), lambda i: (0, 0)),
                pl.BlockSpec((Z, C), lambda i: (0, 0)),
            ],
            out_specs=pl.BlockSpec((TM, C), lambda i: (i, 0)),
        ),
        compiler_params=pltpu.CompilerParams(dimension_semantics=("parallel",)),
    )(s1, adj_p, b1_p, w2_p)

    out = pl.pallas_call(
        _layer2_kernel,
        out_shape=jax.ShapeDtypeStruct((Np, C), jnp.float32),
        grid_spec=pltpu.PrefetchScalarGridSpec(
            num_scalar_prefetch=0,
            grid=(Np // TM2,),
            in_specs=[
                pl.BlockSpec((Np, C), lambda i: (0, 0)),
                pl.BlockSpec((TM2, Np), lambda i: (i, 0)),
                pl.BlockSpec((1, C), lambda i: (0, 0)),
            ],
            out_specs=pl.BlockSpec((TM2, C), lambda i: (i, 0)),
        ),
        compiler_params=pltpu.CompilerParams(dimension_semantics=("parallel",)),
    )(s2, adj_p, b2_p)

    return out


def kernel(x, adj_p, w1_p, b1_p, w2_p, b2_p):
    N = x.shape[0]
    C = w2_p.shape[1]
    out = _forward(x, adj_p, w1_p, b1_p, w2_p, b2_p)
    return out[:N, :C]
